# baseline (speedup 1.0000x reference)
import functools

import jax
import jax.numpy as jnp
from jax import lax
from jax.experimental import pallas as pl
from jax.experimental.pallas import tpu as pltpu
from jax.experimental.pallas import tpu_sc as plsc

_HIDDEN = 1024
_NC = 2
_NS = 16
_NW = _NC * _NS
_L = 16
_NPAIR = 64


@functools.cache
def _sc_lookup(total: int, n_rows: int):
    per_w = total // _NW
    pairs_w = per_w // 2
    ppt = _NPAIR // _NS
    nblk = _HIDDEN // _L
    mesh = plsc.VectorSubcoreMesh(core_axis_name="c", subcore_axis_name="s")

    @functools.partial(
        pl.kernel,
        out_type=jax.ShapeDtypeStruct((total // 2, 2, _HIDDEN), jnp.float32),
        mesh=mesh,
        compiler_params=pltpu.CompilerParams(
            use_tc_tiling_on_sc=False, needs_layout_passes=False
        ),
        scratch_types=[
            pltpu.VMEM((per_w,), jnp.int32),
            pltpu.VMEM((pairs_w,), jnp.int32),
            pltpu.VMEM((n_rows, _HIDDEN), jnp.float32),
            pltpu.VMEM((ppt, 2, _HIDDEN), jnp.float32),
            pltpu.VMEM_SHARED((_NPAIR, 2, _HIDDEN), jnp.float32),
            pltpu.SemaphoreType.DMA,
            pltpu.SemaphoreType.DMA,
        ],
    )
    def k(ids_hbm, table_hbm, out_hbm, idx_v, pid_v, table_v, stage_v,
          pair_sp, tsem, rsem):
        cid = lax.axis_index("c")
        sid = lax.axis_index("s")
        wid = sid * _NC + cid
        base = wid * pairs_w
        cp_t = pltpu.async_copy(table_hbm, table_v, tsem)
        pltpu.sync_copy(ids_hbm.at[wid], idx_v)
        cp_t.wait()
        iota = lax.iota(jnp.int32, _L)

        for i in range(ppt):
            pid = sid * ppt + i
            a = pid // n_rows
            b = pid % n_rows

            def build_blk(t, _, i=i, a=a, b=b):
                off = t * _L
                stage_v[i, 0, pl.ds(off, _L)] = table_v[a, pl.ds(off, _L)]
                stage_v[i, 1, pl.ds(off, _L)] = table_v[b, pl.ds(off, _L)]
                return ()

            lax.fori_loop(0, nblk, build_blk, (), unroll=False)
        pltpu.sync_copy(stage_v, pair_sp.at[pl.ds(sid * ppt, ppt)])
        plsc.subcore_barrier()

        def pid_blk(g, _):
            ev = plsc.load_gather(idx_v, [iota * 2 + g * (2 * _L)])
            od = plsc.load_gather(idx_v, [iota * 2 + g * (2 * _L) + 1])
            pid_v[pl.ds(g * _L, _L)] = ev * n_rows + od
            return ()

        lax.fori_loop(0, pairs_w // _L, pid_blk, (), unroll=False)

        def fire(p, _):
            vec = pid_v[pl.ds((p // _L) * _L, _L)]
            pid = jnp.max(jnp.where(iota == p % _L, vec, 0))
            pltpu.async_copy(pair_sp.at[pid], out_hbm.at[base + p], rsem)
            return ()

        lax.fori_loop(0, pairs_w, fire, (), unroll=False)

        def drain(p, _):
            pltpu.make_async_copy(
                pair_sp.at[0], out_hbm.at[base], rsem
            ).wait()
            return ()

        lax.fori_loop(0, pairs_w, drain, (), unroll=False)

    return k


def kernel(ids, indicator_embs):
    b, s = ids.shape
    total = b * s
    ids_w = ids.astype(jnp.int32).reshape(_NW, total // _NW)
    out = _sc_lookup(total, indicator_embs.shape[0])(ids_w, indicator_embs)
    return out.reshape(b, s, _HIDDEN)


# speedup vs baseline: 1.0753x; 1.0753x over previous
import functools

import jax
import jax.numpy as jnp
from jax import lax
from jax.experimental import pallas as pl
from jax.experimental.pallas import tpu as pltpu
from jax.experimental.pallas import tpu_sc as plsc

_HIDDEN = 1024
_NC = 2
_NS = 16
_NW = _NC * _NS
_L = 16
_NPAIR = 64
_KSPLIT = 128


@functools.cache
def _sc_lookup(total: int, n_rows: int):
    per_w = total // _NW
    pairs_w = per_w // 2
    ppt = _NPAIR // _NS
    nblk = _HIDDEN // _L
    mesh = plsc.VectorSubcoreMesh(core_axis_name="c", subcore_axis_name="s")

    @functools.partial(
        pl.kernel,
        out_type=jax.ShapeDtypeStruct((total // 2, 2, _HIDDEN), jnp.float32),
        mesh=mesh,
        compiler_params=pltpu.CompilerParams(
            use_tc_tiling_on_sc=False, needs_layout_passes=False
        ),
        scratch_types=[
            pltpu.VMEM((per_w,), jnp.int32),
            pltpu.VMEM((pairs_w,), jnp.int32),
            pltpu.VMEM((n_rows, _HIDDEN), jnp.float32),
            pltpu.VMEM((ppt, 2, _HIDDEN), jnp.float32),
            pltpu.VMEM_SHARED((_NPAIR, 2, _HIDDEN), jnp.float32),
            pltpu.SemaphoreType.DMA,
            pltpu.SemaphoreType.DMA,
        ],
    )
    def k(ids_hbm, table_hbm, out_hbm, idx_v, pid_v, table_v, stage_v,
          pair_sp, tsem, rsem):
        cid = lax.axis_index("c")
        sid = lax.axis_index("s")
        wid = sid * _NC + cid
        base = wid * pairs_w
        cp_t = pltpu.async_copy(table_hbm, table_v, tsem)
        pltpu.sync_copy(ids_hbm.at[wid], idx_v)
        cp_t.wait()
        iota = lax.iota(jnp.int32, _L)

        for i in range(ppt):
            pid = sid * ppt + i
            a = pid // n_rows
            b = pid % n_rows

            def build_blk(t, _, i=i, a=a, b=b):
                off = t * _L
                stage_v[i, 0, pl.ds(off, _L)] = table_v[a, pl.ds(off, _L)]
                stage_v[i, 1, pl.ds(off, _L)] = table_v[b, pl.ds(off, _L)]
                return ()

            lax.fori_loop(0, nblk, build_blk, (), unroll=False)
        pltpu.sync_copy(stage_v, pair_sp.at[pl.ds(sid * ppt, ppt)])
        plsc.subcore_barrier()

        def pid_blk(g, _):
            ev = plsc.load_gather(idx_v, [iota * 2 + g * (2 * _L)])
            od = plsc.load_gather(idx_v, [iota * 2 + g * (2 * _L) + 1])
            pid_v[pl.ds(g * _L, _L)] = ev * n_rows + od
            return ()

        lax.fori_loop(0, pairs_w // _L, pid_blk, (), unroll=False)

        def fire_pair(p, _):
            vec = pid_v[pl.ds((p // _L) * _L, _L)]
            pid = jnp.max(jnp.where(iota == p % _L, vec, 0))
            pltpu.async_copy(pair_sp.at[pid], out_hbm.at[base + p], rsem)
            return ()

        def fire_rows(p, _):
            vec = pid_v[pl.ds((p // _L) * _L, _L)]
            pid = jnp.max(jnp.where(iota == p % _L, vec, 0))
            a = pid // n_rows
            b = pid % n_rows
            pltpu.async_copy(table_v.at[a], out_hbm.at[base + p, 0], tsem)
            pltpu.async_copy(table_v.at[b], out_hbm.at[base + p, 1], tsem)
            return ()

        lax.fori_loop(0, _KSPLIT, fire_pair, (), unroll=False)
        lax.fori_loop(_KSPLIT, pairs_w, fire_rows, (), unroll=False)

        def drain_pair(p, _):
            pltpu.make_async_copy(
                pair_sp.at[0], out_hbm.at[base], rsem
            ).wait()
            return ()

        def drain_rows(p, _):
            pltpu.make_async_copy(
                table_v.at[0], out_hbm.at[base, 0], tsem
            ).wait()
            return ()

        lax.fori_loop(0, _KSPLIT, drain_pair, (), unroll=False)
        lax.fori_loop(0, 2 * (pairs_w - _KSPLIT), drain_rows, (), unroll=False)

    return k


def kernel(ids, indicator_embs):
    b, s = ids.shape
    total = b * s
    ids_w = ids.astype(jnp.int32).reshape(_NW, total // _NW)
    out = _sc_lookup(total, indicator_embs.shape[0])(ids_w, indicator_embs)
    return out.reshape(b, s, _HIDDEN)
